# baseline (speedup 1.0000x reference)
import functools

import jax
import jax.numpy as jnp
from jax import lax
from jax.experimental import pallas as pl
from jax.experimental.pallas import tpu as pltpu
from jax.experimental.pallas import tpu_sc as plsc

N_NODES = 10000
D = 128
DEXT = 144
N_PAD = 10240
CHUNK = 128
N_SUBCORES = 16
ROWS_PER_TILE = N_PAD // N_SUBCORES
ROW_CHUNKS = ROWS_PER_TILE // CHUNK
CHUNKS_PER_TILE = 160
PAIRS = CHUNKS_PER_TILE // 2
EDGES_PER_TILE = CHUNKS_PER_TILE * CHUNK
E_PAD = EDGES_PER_TILE * N_SUBCORES

BLK = 1024


def _sc_body(featx_hbm, idxpk_hbm, sin_hbm, sout_hbm,
             pairA, pairB, rows0, rows1, accum, gsems, ssems, isems):
    c = lax.axis_index("c")
    s = lax.axis_index("s")
    tile_row0 = s * ROWS_PER_TILE

    zeros16 = jnp.zeros((16,), jnp.float32)

    def zrow(i, _):
        def zcol(j, _):
            rows0[i, pl.ds(j * 16, 16)] = zeros16
            return 0
        return lax.fori_loop(0, DEXT // 16, zcol, 0)

    lax.fori_loop(0, CHUNK, zrow, 0)

    def zacc(j, _):
        pltpu.sync_copy(rows0, accum.at[pl.ds(tile_row0 + j * CHUNK, CHUNK)])
        return 0

    lax.fori_loop(0, ROW_CHUNKS, zacc, 0)

    def direction(gslot, sslot, out_hbm):
        bufs = ((pairA, rows0, 0), (pairB, rows1, 1))

        def fire_idx(i, bs):
            pair, _, b = bs
            pltpu.async_copy(idxpk_hbm.at[s, i], pair, isems[b])

        def wait_idx(bs):
            pair, _, b = bs
            pltpu.make_async_copy(idxpk_hbm.at[s, 0], pair, isems[b]).wait()

        def fire_gather(bs):
            pair, rows, b = bs
            pltpu.async_copy(featx_hbm.at[pair.at[gslot]], rows, gsems[b])

        def wait_gather(bs):
            pair, rows, b = bs
            pltpu.make_async_copy(featx_hbm.at[pair.at[gslot]], rows,
                                  gsems[b]).wait()

        def fire_scatter(bs):
            pair, rows, b = bs
            pltpu.async_copy(rows, accum.at[pair.at[sslot]], ssems[b],
                             add=True)

        def wait_scatter(bs):
            pair, rows, b = bs
            pltpu.make_async_copy(rows, accum.at[pair.at[sslot]],
                                  ssems[b]).wait()

        A, B = bufs

        def half(nidx, X, Y, first=False):
            if not first:
                wait_scatter(Y)
            fire_idx(nidx, Y)
            wait_gather(X)
            fire_scatter(X)
            wait_idx(Y)
            fire_gather(Y)

        fire_idx(0, A)
        wait_idx(A)
        fire_gather(A)
        plsc.subcore_barrier()
        half(1, A, B, first=True)
        half(2, B, A)

        def body(k, _):
            i = 2 * k
            half(i + 1, A, B)
            half(i + 2, B, A)
            return 0

        lax.fori_loop(1, PAIRS - 1, body, 0)
        half(PAIRS * 2 - 1, A, B)
        wait_scatter(A)
        wait_gather(B)
        fire_scatter(B)
        wait_scatter(B)
        plsc.subcore_barrier()

        def cout(j, _):
            r0 = tile_row0 + j * CHUNK
            pltpu.sync_copy(accum.at[pl.ds(r0, CHUNK)], out_hbm.at[pl.ds(r0, CHUNK)])
            return 0

        lax.fori_loop(0, ROW_CHUNKS, cout, 0)

    @pl.when(c == 0)
    def _():
        direction(0, 1, sin_hbm)

    @pl.when(c == 1)
    def _():
        direction(1, 0, sout_hbm)


def _segment_sums(featx, idxpk):
    mesh = plsc.VectorSubcoreMesh(core_axis_name="c", subcore_axis_name="s")
    k = pl.kernel(
        _sc_body,
        out_type=(
            jax.ShapeDtypeStruct((N_PAD, DEXT), jnp.float32),
            jax.ShapeDtypeStruct((N_PAD, DEXT), jnp.float32),
        ),
        mesh=mesh,
        scratch_types=[
            pltpu.VMEM((2, CHUNK), jnp.int32),
            pltpu.VMEM((2, CHUNK), jnp.int32),
            pltpu.VMEM((CHUNK, DEXT), jnp.float32),
            pltpu.VMEM((CHUNK, DEXT), jnp.float32),
            pltpu.VMEM_SHARED((N_PAD, DEXT), jnp.float32),
            [pltpu.SemaphoreType.DMA] * 2,
            [pltpu.SemaphoreType.DMA] * 2,
            [pltpu.SemaphoreType.DMA] * 2,
        ],
        compiler_params=pltpu.CompilerParams(use_tc_tiling_on_sc=False),
    )
    return k(featx, idxpk)


def _tc_body(featx, sin, sout, w_in, b_in, w_out, b_out, w_ih, w_hh,
             b_ih, b_hh, out):
    f = featx[...][:, :D]
    si = sin[...]
    so = sout[...]
    deg_i = si[:, D:D + 1]
    deg_o = so[:, D:D + 1]
    mean_i = si[:, :D] / jnp.maximum(deg_i, 1.0)
    mean_o = so[:, :D] / jnp.maximum(deg_o, 1.0)
    m_i = jnp.minimum(deg_i, 1.0)
    m_o = jnp.minimum(deg_o, 1.0)

    def dotT(x, w):
        return lax.dot_general(x, w, (((1,), (1,)), ((), ())),
                               preferred_element_type=jnp.float32)

    a_i = dotT(mean_i, w_in[...]) + m_i * b_in[...]
    a_o = dotT(mean_o, w_out[...]) + m_o * b_out[...]
    wih = w_ih[...]
    gi = dotT(a_i, wih[:, :D]) + dotT(a_o, wih[:, D:]) + b_ih[...]
    gh = dotT(f, w_hh[...]) + b_hh[...]
    r = jax.nn.sigmoid(gi[:, :D] + gh[:, :D])
    z = jax.nn.sigmoid(gi[:, D:2 * D] + gh[:, D:2 * D])
    n = jnp.tanh(gi[:, 2 * D:] + r * gh[:, 2 * D:])
    out[...] = (1.0 - z) * n + z * f


def _gru_update(featx, sin, sout, W_in, b_in, W_out, b_out, W_ih, W_hh,
                b_ih, b_hh):
    grid = N_PAD // BLK
    row_spec = lambda shape: pl.BlockSpec((BLK, shape), lambda i: (i, 0))
    full = lambda s: pl.BlockSpec(s, lambda i: (0,) * len(s))
    return pl.pallas_call(
        _tc_body,
        grid=(grid,),
        in_specs=[
            row_spec(DEXT),
            row_spec(DEXT),
            row_spec(DEXT),
            full((D, D)),
            full((1, D)),
            full((D, D)),
            full((1, D)),
            full((3 * D, 2 * D)),
            full((3 * D, D)),
            full((1, 3 * D)),
            full((1, 3 * D)),
        ],
        out_specs=row_spec(D),
        out_shape=jax.ShapeDtypeStruct((N_PAD, D), jnp.float32),
    )(featx, sin, sout, W_in, b_in.reshape(1, D), W_out,
      b_out.reshape(1, D), W_ih, W_hh, b_ih.reshape(1, 3 * D),
      b_hh.reshape(1, 3 * D))


@jax.jit
def kernel(feat, edge_index, W_in, b_in, W_out, b_out, W_ih, W_hh, b_ih, b_hh):
    n = feat.shape[0]
    src = edge_index[0].astype(jnp.int32)
    dst = edge_index[1].astype(jnp.int32)
    e = src.shape[0]
    padlen = E_PAD - e
    fill = jnp.full((padlen,), n, jnp.int32)
    srcp = jnp.concatenate([src, fill]).reshape(N_SUBCORES, CHUNKS_PER_TILE, CHUNK)
    dstp = jnp.concatenate([dst, fill]).reshape(N_SUBCORES, CHUNKS_PER_TILE, CHUNK)
    idxpk = jnp.stack([srcp, dstp], axis=2)

    featx = jnp.zeros((N_PAD, DEXT), jnp.float32)
    featx = featx.at[:n, :D].set(feat)
    featx = featx.at[:n, D].set(1.0)

    sin, sout = _segment_sums(featx, idxpk)
    hn = _gru_update(featx, sin, sout, W_in, b_in, W_out, b_out, W_ih,
                     W_hh, b_ih, b_hh)
    return hn[:n]
